# Initial kernel scaffold; baseline (speedup 1.0000x reference)
import jax
import jax.numpy as jnp
from jax.experimental import pallas as pl


def kernel(pred_logits, pred_boxes, target_sizes):
    raise NotImplementedError("write your pallas kernel here")


# speedup vs baseline: 12.4844x; 12.4844x over previous
import functools

import jax
import jax.numpy as jnp
from jax.experimental import pallas as pl

_NQ = 1000
_NC = 91
_PRE_TOPK = 10000
_KEEP = 100
_IOU_THR = 0.7
_BIG = 1 << 30


def _nms_body(logits_ref, boxes_ref, scale_ref, s_out_ref, l_out_ref, b_out_ref):
    logits = logits_ref[0]
    s = jax.nn.sigmoid(logits)
    sbits = jax.lax.bitcast_convert_type(s, jnp.int32)

    row_iota = jax.lax.broadcasted_iota(jnp.int32, (_NC, _NQ), 0)
    col_iota = jax.lax.broadcasted_iota(jnp.int32, (_NC, _NQ), 1)
    fidx = col_iota * _NC + row_iota

    sw = jnp.sum(scale_ref[0, 0:1, 0:1])
    sh = jnp.sum(scale_ref[0, 0:1, 1:2])
    cx = boxes_ref[0, 0:1, :]
    cy = boxes_ref[0, 1:2, :]
    w = boxes_ref[0, 2:3, :]
    h = boxes_ref[0, 3:4, :]
    x1s = (cx - 0.5 * w) * sw
    y1s = (cy - 0.5 * h) * sh
    x2s = (cx + 0.5 * w) * sw
    y2s = (cy + 0.5 * h) * sh

    def cnt_gt(t):
        return jnp.sum((sbits > t).astype(jnp.int32))

    def bs_val(_, lohi):
        lo, hi = lohi
        mid = (lo + hi) // 2
        c = cnt_gt(mid)
        take_hi = c >= _PRE_TOPK
        return (jnp.where(take_hi, mid, lo), jnp.where(take_hi, hi, mid))

    lo0 = jnp.int32(-1)
    hi0 = jnp.int32(0x3F800000)
    _, tau = jax.lax.fori_loop(0, 31, bs_val, (lo0, hi0))
    n_gt = cnt_gt(tau)
    need = _PRE_TOPK - n_gt
    is_tie = sbits == tau

    def cnt_tie_lt(m):
        return jnp.sum((is_tie & (fidx < m)).astype(jnp.int32))

    def bs_idx(_, lohi):
        lo, hi = lohi
        mid = (lo + hi) // 2
        c = cnt_tie_lt(mid)
        take_hi = c >= need
        return (jnp.where(take_hi, lo, mid), jnp.where(take_hi, mid, hi))

    _, mstar = jax.lax.fori_loop(0, 17, bs_idx, (jnp.int32(0), jnp.int32(_NQ * _NC)))
    elig = (sbits > tau) | (is_tie & (fidx < mstar))

    row_any = jnp.any(elig, axis=0, keepdims=True)
    qmax = jnp.maximum(jnp.maximum(x1s, x2s), jnp.maximum(y1s, y2s))
    neg_big = jnp.float32(-3.4e38)
    maxc = jnp.max(jnp.where(row_any, qmax, neg_big))
    off_unit = maxc + 1.0
    offv = row_iota[:, 0:1].astype(jnp.float32) * off_unit

    cox1 = x1s + offv
    coy1 = y1s + offv
    cox2 = x2s + offv
    coy2 = y2s + offv
    carea = jnp.maximum(cox2 - cox1, 0.0) * jnp.maximum(coy2 - coy1, 0.0)

    t1_bits = jnp.max(sbits)
    t1_fidx = jnp.min(jnp.where(sbits == t1_bits, fidx, _BIG))

    def extract_q(vec_1xnq, q):
        return jnp.sum(jnp.where(col_iota[0:1, :] == q, vec_1xnq, 0.0))

    def pick_geometry(pidx):
        q = pidx // _NC
        c = pidx % _NC
        o = c.astype(jnp.float32) * off_unit
        px1 = extract_q(x1s, q)
        py1 = extract_q(y1s, q)
        px2 = extract_q(x2s, q)
        py2 = extract_q(y2s, q)
        return q, c, o, px1, py1, px2, py2

    lane_iota = jax.lax.broadcasted_iota(jnp.int32, (1, 128), 1)

    def body(i, carry):
        msb, ks, kl, kx1, ky1, kx2, ky2 = carry
        mb = jnp.max(msb)
        empty = mb < 0
        pidx_direct = jnp.min(jnp.where(msb == mb, fidx, _BIG))
        pidx = jnp.where(empty, t1_fidx, pidx_direct)
        sval = jnp.sum(jnp.where(fidx == pidx, s, 0.0))
        q, c, o, px1, py1, px2, py2 = pick_geometry(pidx)
        pox1 = px1 + o
        poy1 = py1 + o
        pox2 = px2 + o
        poy2 = py2 + o
        parea = jnp.maximum(pox2 - pox1, 0.0) * jnp.maximum(poy2 - poy1, 0.0)
        ix = jnp.maximum(jnp.minimum(pox2, cox2) - jnp.maximum(pox1, cox1), 0.0)
        iy = jnp.maximum(jnp.minimum(poy2, coy2) - jnp.maximum(poy1, coy1), 0.0)
        inter = ix * iy
        union = parea + carea - inter
        iou = inter / jnp.maximum(union, 1e-9)
        msb = jnp.where(iou > _IOU_THR, jnp.int32(-1), msb)
        cm = lane_iota == i
        ks = jnp.where(cm, sval, ks)
        kl = jnp.where(cm, c, kl)
        kx1 = jnp.where(cm, px1, kx1)
        ky1 = jnp.where(cm, py1, ky1)
        kx2 = jnp.where(cm, px2, kx2)
        ky2 = jnp.where(cm, py2, ky2)
        return msb, ks, kl, kx1, ky1, kx2, ky2

    zf = jnp.zeros((1, 128), jnp.float32)
    zi = jnp.zeros((1, 128), jnp.int32)
    msb0 = jnp.where(elig, sbits, jnp.int32(-1))
    carry0 = (msb0, zf, zi, zf, zf, zf, zf)
    _, ks, kl, kx1, ky1, kx2, ky2 = jax.lax.fori_loop(0, _KEEP, body, carry0)

    s_out_ref[0, 0:1, :] = ks[:, :_KEEP]
    l_out_ref[0, 0:1, :] = kl[:, :_KEEP]
    b_out_ref[0, 0:1, :] = kx1[:, :_KEEP]
    b_out_ref[0, 1:2, :] = ky1[:, :_KEEP]
    b_out_ref[0, 2:3, :] = kx2[:, :_KEEP]
    b_out_ref[0, 3:4, :] = ky2[:, :_KEEP]


def _build_call(interpret=False):
    return pl.pallas_call(
        _nms_body,
        grid=(4,),
        in_specs=[
            pl.BlockSpec((1, _NC, _NQ), lambda b: (b, 0, 0)),
            pl.BlockSpec((1, 4, _NQ), lambda b: (b, 0, 0)),
            pl.BlockSpec((1, 1, 4), lambda b: (b, 0, 0)),
        ],
        out_specs=[
            pl.BlockSpec((1, 1, _KEEP), lambda b: (b, 0, 0)),
            pl.BlockSpec((1, 1, _KEEP), lambda b: (b, 0, 0)),
            pl.BlockSpec((1, 4, _KEEP), lambda b: (b, 0, 0)),
        ],
        out_shape=[
            jax.ShapeDtypeStruct((4, 1, _KEEP), jnp.float32),
            jax.ShapeDtypeStruct((4, 1, _KEEP), jnp.int32),
            jax.ShapeDtypeStruct((4, 4, _KEEP), jnp.float32),
        ],
        interpret=interpret,
    )


@jax.jit
def kernel(pred_logits, pred_boxes, target_sizes):
    logits_t = jnp.transpose(pred_logits, (0, 2, 1))
    boxes_t = jnp.transpose(pred_boxes, (0, 2, 1))
    img_h = target_sizes[:, 0].astype(jnp.float32)
    img_w = target_sizes[:, 1].astype(jnp.float32)
    scale = jnp.stack([img_w, img_h, img_w, img_h], axis=1)
    scale = scale.reshape(4, 1, 4)
    scores, labels, boxes_cm = _build_call()(logits_t, boxes_t, scale)
    return (scores.reshape(4, _KEEP), labels.reshape(4, _KEEP),
            jnp.transpose(boxes_cm, (0, 2, 1)))


# speedup vs baseline: 13.2491x; 1.0613x over previous
import functools

import jax
import jax.numpy as jnp
from jax.experimental import pallas as pl
from jax.experimental.pallas import tpu as pltpu

_NQ = 1000
_NC = 91
_PRE_TOPK = 10000
_KEEP = 100
_IOU_THR = 0.7
_BIG = 1 << 30


def _nms_body(logits_ref, boxes_t_ref, boxes_raw_ref, scale_ref,
              s_out_ref, l_out_ref, b_out_ref,
              msb_ref, s_ref, rmax_ref, rbest_ref):
    logits = logits_ref[0]
    s = jax.nn.sigmoid(logits)
    s_ref[...] = s
    sbits = jax.lax.bitcast_convert_type(s, jnp.int32)

    row_iota = jax.lax.broadcasted_iota(jnp.int32, (_NC, _NQ), 0)
    col_iota = jax.lax.broadcasted_iota(jnp.int32, (_NC, _NQ), 1)
    fidx = col_iota * _NC + row_iota

    def cnt_gt(t):
        return jnp.sum((sbits > t).astype(jnp.int32))

    def bs_val(_, lohi):
        lo, hi = lohi
        mid = (lo + hi) // 2
        take_hi = cnt_gt(mid) >= _PRE_TOPK
        return (jnp.where(take_hi, mid, lo), jnp.where(take_hi, hi, mid))

    _, tau = jax.lax.fori_loop(0, 31, bs_val, (jnp.int32(-1), jnp.int32(0x3F800000)))
    need = _PRE_TOPK - cnt_gt(tau)
    is_tie = sbits == tau

    def bs_idx(_, lohi):
        lo, hi = lohi
        mid = (lo + hi) // 2
        take_hi = jnp.sum((is_tie & (fidx < mid)).astype(jnp.int32)) >= need
        return (jnp.where(take_hi, lo, mid), jnp.where(take_hi, mid, hi))

    _, mstar = jax.lax.fori_loop(0, 17, bs_idx, (jnp.int32(0), jnp.int32(_NQ * _NC)))
    elig = (sbits > tau) | (is_tie & (fidx < mstar))

    sw = jnp.sum(scale_ref[0, 0:1, 0:1])
    sh = jnp.sum(scale_ref[0, 0:1, 1:2])
    cxt = boxes_t_ref[0, 0:1, :]
    cyt = boxes_t_ref[0, 1:2, :]
    wt = boxes_t_ref[0, 2:3, :]
    ht = boxes_t_ref[0, 3:4, :]
    x1t = (cxt - 0.5 * wt) * sw
    y1t = (cyt - 0.5 * ht) * sh
    x2t = (cxt + 0.5 * wt) * sw
    y2t = (cyt + 0.5 * ht) * sh
    qmax = jnp.maximum(jnp.maximum(x1t, x2t), jnp.maximum(y1t, y2t))
    row_any = jnp.any(elig, axis=0, keepdims=True)
    maxc = jnp.max(jnp.where(row_any, qmax, jnp.float32(-3.4e38)))
    off_unit = maxc + 1.0

    msb0 = jnp.where(elig, sbits, jnp.int32(-1))
    msb_ref[...] = msb0
    rmax0 = jnp.max(msb0, axis=1, keepdims=True)
    qmin0 = jnp.min(jnp.where(msb0 == rmax0, col_iota, _BIG),
                    axis=1, keepdims=True)
    riota = jax.lax.broadcasted_iota(jnp.int32, (_NC, 1), 0)
    rmax_ref[...] = rmax0
    rbest_ref[...] = qmin0 * _NC + riota

    lane_iota = jax.lax.broadcasted_iota(jnp.int32, (1, 128), 1)
    col1 = jax.lax.broadcasted_iota(jnp.int32, (1, _NQ), 1)

    def cond(carry):
        count, done = carry[0], carry[1]
        return (count < _KEEP) & (done == 0)

    def body(carry):
        (count, done, ks, kl, kbx1, kby1, kbx2, kby2,
         kx1o, ky1o, kx2o, ky2o, karea) = carry
        rm = rmax_ref[...]
        mb = jnp.max(rm)
        newdone = (mb < 0).astype(jnp.int32)
        fpick = jnp.min(jnp.where(rm == mb, rbest_ref[...], _BIG))
        q = fpick // _NC
        c = fpick % _NC
        row = msb_ref[pl.ds(c, 1), :]
        row = jnp.where(col1 == q, jnp.int32(-1), row)
        msb_ref[pl.ds(c, 1), :] = row
        nrmax = jnp.max(row)
        nqmin = jnp.min(jnp.where(row == nrmax, col1, _BIG))
        rmax_ref[pl.ds(c, 1), :] = jnp.broadcast_to(nrmax, (1, 1))
        rbest_ref[pl.ds(c, 1), :] = jnp.broadcast_to(nqmin * _NC + c, (1, 1))
        srow = s_ref[pl.ds(c, 1), :]
        sval = jnp.sum(jnp.where(col1 == q, srow, 0.0))
        braw = boxes_raw_ref[0, pl.ds(q, 1), :]
        bcx = jnp.sum(braw[:, 0:1])
        bcy = jnp.sum(braw[:, 1:2])
        bw = jnp.sum(braw[:, 2:3])
        bh = jnp.sum(braw[:, 3:4])
        px1 = (bcx - 0.5 * bw) * sw
        py1 = (bcy - 0.5 * bh) * sh
        px2 = (bcx + 0.5 * bw) * sw
        py2 = (bcy + 0.5 * bh) * sh
        o = c.astype(jnp.float32) * off_unit
        cox1 = px1 + o
        coy1 = py1 + o
        cox2 = px2 + o
        coy2 = py2 + o
        carea_s = jnp.maximum(cox2 - cox1, 0.0) * jnp.maximum(coy2 - coy1, 0.0)
        ix = jnp.maximum(jnp.minimum(kx2o, cox2) - jnp.maximum(kx1o, cox1), 0.0)
        iy = jnp.maximum(jnp.minimum(ky2o, coy2) - jnp.maximum(ky1o, coy1), 0.0)
        inter = ix * iy
        union = karea + carea_s - inter
        iou = inter / jnp.maximum(union, 1e-9)
        supp = jnp.max(iou) > _IOU_THR
        keepit = jnp.logical_not(supp) & (newdone == 0)
        cm = (lane_iota == count) & keepit
        ks = jnp.where(cm, sval, ks)
        kl = jnp.where(cm, c, kl)
        kbx1 = jnp.where(cm, px1, kbx1)
        kby1 = jnp.where(cm, py1, kby1)
        kbx2 = jnp.where(cm, px2, kbx2)
        kby2 = jnp.where(cm, py2, kby2)
        kx1o = jnp.where(cm, cox1, kx1o)
        ky1o = jnp.where(cm, coy1, ky1o)
        kx2o = jnp.where(cm, cox2, kx2o)
        ky2o = jnp.where(cm, coy2, ky2o)
        karea = jnp.where(cm, carea_s, karea)
        count = count + keepit.astype(jnp.int32)
        return (count, newdone, ks, kl, kbx1, kby1, kbx2, kby2,
                kx1o, ky1o, kx2o, ky2o, karea)

    zf = jnp.zeros((1, 128), jnp.float32)
    zi = jnp.zeros((1, 128), jnp.int32)
    carry0 = (jnp.int32(0), jnp.int32(0), zf, zi, zf, zf, zf, zf,
              zf, zf, zf, zf, zf)
    (count, _, ks, kl, kbx1, kby1, kbx2, kby2,
     _, _, _, _, _) = jax.lax.while_loop(cond, body, carry0)

    first = lane_iota == 0
    padm = lane_iota >= count
    ks = jnp.where(padm, jnp.sum(jnp.where(first, ks, 0.0)), ks)
    kl = jnp.where(padm, jnp.sum(jnp.where(first, kl, 0)), kl)
    kbx1 = jnp.where(padm, jnp.sum(jnp.where(first, kbx1, 0.0)), kbx1)
    kby1 = jnp.where(padm, jnp.sum(jnp.where(first, kby1, 0.0)), kby1)
    kbx2 = jnp.where(padm, jnp.sum(jnp.where(first, kbx2, 0.0)), kbx2)
    kby2 = jnp.where(padm, jnp.sum(jnp.where(first, kby2, 0.0)), kby2)

    s_out_ref[0, 0:1, :] = ks[:, :_KEEP]
    l_out_ref[0, 0:1, :] = kl[:, :_KEEP]
    b_out_ref[0, 0:1, :] = kbx1[:, :_KEEP]
    b_out_ref[0, 1:2, :] = kby1[:, :_KEEP]
    b_out_ref[0, 2:3, :] = kbx2[:, :_KEEP]
    b_out_ref[0, 3:4, :] = kby2[:, :_KEEP]


def _build_call(interpret=False):
    return pl.pallas_call(
        _nms_body,
        grid=(4,),
        in_specs=[
            pl.BlockSpec((1, _NC, _NQ), lambda b: (b, 0, 0)),
            pl.BlockSpec((1, 4, _NQ), lambda b: (b, 0, 0)),
            pl.BlockSpec((1, _NQ, 4), lambda b: (b, 0, 0)),
            pl.BlockSpec((1, 1, 4), lambda b: (b, 0, 0)),
        ],
        out_specs=[
            pl.BlockSpec((1, 1, _KEEP), lambda b: (b, 0, 0)),
            pl.BlockSpec((1, 1, _KEEP), lambda b: (b, 0, 0)),
            pl.BlockSpec((1, 4, _KEEP), lambda b: (b, 0, 0)),
        ],
        out_shape=[
            jax.ShapeDtypeStruct((4, 1, _KEEP), jnp.float32),
            jax.ShapeDtypeStruct((4, 1, _KEEP), jnp.int32),
            jax.ShapeDtypeStruct((4, 4, _KEEP), jnp.float32),
        ],
        scratch_shapes=[
            pltpu.VMEM((_NC, _NQ), jnp.int32),
            pltpu.VMEM((_NC, _NQ), jnp.float32),
            pltpu.VMEM((_NC, 1), jnp.int32),
            pltpu.VMEM((_NC, 1), jnp.int32),
        ],
        interpret=interpret,
    )


@jax.jit
def kernel(pred_logits, pred_boxes, target_sizes):
    logits_t = jnp.transpose(pred_logits, (0, 2, 1))
    boxes_t = jnp.transpose(pred_boxes, (0, 2, 1))
    img_h = target_sizes[:, 0].astype(jnp.float32)
    img_w = target_sizes[:, 1].astype(jnp.float32)
    scale = jnp.stack([img_w, img_h, img_w, img_h], axis=1).reshape(4, 1, 4)
    scores, labels, boxes_cm = _build_call()(logits_t, boxes_t, pred_boxes, scale)
    return (scores.reshape(4, _KEEP), labels.reshape(4, _KEEP),
            jnp.transpose(boxes_cm, (0, 2, 1)))


# speedup vs baseline: 25.1325x; 1.8969x over previous
import functools

import jax
import jax.numpy as jnp
from jax.experimental import pallas as pl
from jax.experimental.pallas import tpu as pltpu

_BS = 4
_NQ = 1000
_NC = 91
_PRE_TOPK = 10000
_KEEP = 100
_IOU_THR = 0.7
_BIG = 1 << 30


def _nms_body(logits_ref, boxes_t_ref, boxes_raw_ref, scale_ref,
              s_out_ref, l_out_ref, x1_out_ref, y1_out_ref, x2_out_ref,
              y2_out_ref, msb_ref, s_ref, rmax_ref, rbest_ref):
    row_iota = jax.lax.broadcasted_iota(jnp.int32, (_NC, _NQ), 0)
    col_iota = jax.lax.broadcasted_iota(jnp.int32, (_NC, _NQ), 1)
    fidx = col_iota * _NC + row_iota
    riota = jax.lax.broadcasted_iota(jnp.int32, (_NC, 1), 0)
    col1 = jax.lax.broadcasted_iota(jnp.int32, (1, _NQ), 1)
    lane_iota = jax.lax.broadcasted_iota(jnp.int32, (_BS, 128), 1)
    batch_col = jax.lax.broadcasted_iota(jnp.int32, (1, _BS), 1)

    sbits_all = []
    sw_all = []
    sh_all = []
    for b in range(_BS):
        s_b = jax.nn.sigmoid(logits_ref[b])
        s_ref[b * _NC:(b + 1) * _NC, :] = s_b
        sbits_all.append(jax.lax.bitcast_convert_type(s_b, jnp.int32))
        sw_all.append(jnp.sum(scale_ref[b:b + 1, 0:1]))
        sh_all.append(jnp.sum(scale_ref[b:b + 1, 1:2]))

    def bs_val(_, state):
        out = []
        for b in range(_BS):
            lo, hi = state[2 * b], state[2 * b + 1]
            mid = (lo + hi) // 2
            take_hi = jnp.sum((sbits_all[b] > mid).astype(jnp.int32)) >= _PRE_TOPK
            out.append(jnp.where(take_hi, mid, lo))
            out.append(jnp.where(take_hi, hi, mid))
        return tuple(out)

    st0 = (jnp.int32(-1), jnp.int32(0x3F800000)) * _BS
    st = jax.lax.fori_loop(0, 31, bs_val, st0)
    taus = [st[2 * b + 1] for b in range(_BS)]
    needs = [
        _PRE_TOPK - jnp.sum((sbits_all[b] > taus[b]).astype(jnp.int32))
        for b in range(_BS)
    ]
    ties = [sbits_all[b] == taus[b] for b in range(_BS)]

    def bs_idx(_, state):
        out = []
        for b in range(_BS):
            lo, hi = state[2 * b], state[2 * b + 1]
            mid = (lo + hi) // 2
            cnt = jnp.sum((ties[b] & (fidx < mid)).astype(jnp.int32))
            take_hi = cnt >= needs[b]
            out.append(jnp.where(take_hi, lo, mid))
            out.append(jnp.where(take_hi, mid, hi))
        return tuple(out)

    st0 = (jnp.int32(0), jnp.int32(_NQ * _NC)) * _BS
    st = jax.lax.fori_loop(0, 17, bs_idx, st0)
    mstars = [st[2 * b + 1] for b in range(_BS)]

    off_units = []
    for b in range(_BS):
        elig = (sbits_all[b] > taus[b]) | (ties[b] & (fidx < mstars[b]))
        msb0 = jnp.where(elig, sbits_all[b], jnp.int32(-1))
        msb_ref[b * _NC:(b + 1) * _NC, :] = msb0
        rmax0 = jnp.max(msb0, axis=1, keepdims=True)
        qmin0 = jnp.min(jnp.where(msb0 == rmax0, col_iota, _BIG),
                        axis=1, keepdims=True)
        rmax_ref[:, b:b + 1] = rmax0
        rbest_ref[:, b:b + 1] = qmin0 * _NC + riota
        cxt = boxes_t_ref[b, 0:1, :]
        cyt = boxes_t_ref[b, 1:2, :]
        wt = boxes_t_ref[b, 2:3, :]
        ht = boxes_t_ref[b, 3:4, :]
        x1t = (cxt - 0.5 * wt) * sw_all[b]
        y1t = (cyt - 0.5 * ht) * sh_all[b]
        x2t = (cxt + 0.5 * wt) * sw_all[b]
        y2t = (cyt + 0.5 * ht) * sh_all[b]
        qmax = jnp.maximum(jnp.maximum(x1t, x2t), jnp.maximum(y1t, y2t))
        row_any = jnp.any(elig, axis=0, keepdims=True)
        maxc = jnp.max(jnp.where(row_any, qmax, jnp.float32(-3.4e38)))
        off_units.append(maxc + 1.0)

    biota = jax.lax.broadcasted_iota(jnp.int32, (_BS, 1), 0)

    def colvec(vals, zero):
        out = jnp.full((_BS, 1), zero)
        for b in range(_BS):
            out = jnp.where(biota == b, vals[b], out)
        return out

    def cond(carry):
        counts, dones = carry[0], carry[1]
        live = ((counts < _KEEP) & (dones == 0)).astype(jnp.int32)
        return jnp.sum(live) > 0

    def body(carry):
        (counts, dones, ks, kl, kbx1, kby1, kbx2, kby2,
         kx1o, ky1o, kx2o, ky2o, karea) = carry
        rm = rmax_ref[...]
        mb = jnp.max(rm, axis=0, keepdims=True)
        fpick = jnp.min(jnp.where(rm == mb, rbest_ref[...], _BIG),
                        axis=0, keepdims=True)

        svals = []
        cs = []
        mbneg = []
        px1s, py1s, px2s, py2s = [], [], [], []
        cox1s, coy1s, cox2s, coy2s = [], [], [], []
        for b in range(_BS):
            f_b = jnp.sum(jnp.where(batch_col == b, fpick, 0))
            mb_b = jnp.sum(jnp.where(batch_col == b, mb, 0))
            cnt_b = jnp.sum(jnp.where(biota == b, counts, 0))
            done_b = jnp.sum(jnp.where(biota == b, dones, 0))
            live_b = (cnt_b < _KEEP) & (done_b == 0) & (mb_b >= 0)
            mbneg.append((mb_b < 0).astype(jnp.int32))
            q = f_b // _NC
            c = f_b % _NC
            r = b * _NC + c
            row = msb_ref[pl.ds(r, 1), :]
            row = jnp.where((col1 == q) & live_b, jnp.int32(-1), row)
            msb_ref[pl.ds(r, 1), :] = row
            nrmax = jnp.max(row)
            nqmin = jnp.min(jnp.where(row == nrmax, col1, _BIG))
            rmax_ref[pl.ds(c, 1), b:b + 1] = jnp.broadcast_to(nrmax, (1, 1))
            rbest_ref[pl.ds(c, 1), b:b + 1] = jnp.broadcast_to(
                nqmin * _NC + c, (1, 1))
            srow = s_ref[pl.ds(r, 1), :]
            svals.append(jnp.sum(jnp.where(col1 == q, srow, 0.0)))
            braw = boxes_raw_ref[b, pl.ds(q, 1), :]
            bcx = jnp.sum(braw[:, 0:1])
            bcy = jnp.sum(braw[:, 1:2])
            bw = jnp.sum(braw[:, 2:3])
            bh = jnp.sum(braw[:, 3:4])
            px1 = (bcx - 0.5 * bw) * sw_all[b]
            py1 = (bcy - 0.5 * bh) * sh_all[b]
            px2 = (bcx + 0.5 * bw) * sw_all[b]
            py2 = (bcy + 0.5 * bh) * sh_all[b]
            o = c.astype(jnp.float32) * off_units[b]
            cs.append(c)
            px1s.append(px1)
            py1s.append(py1)
            px2s.append(px2)
            py2s.append(py2)
            cox1s.append(px1 + o)
            coy1s.append(py1 + o)
            cox2s.append(px2 + o)
            coy2s.append(py2 + o)

        newdones = dones | colvec(mbneg, jnp.int32(0))
        cox1 = colvec(cox1s, jnp.float32(0))
        coy1 = colvec(coy1s, jnp.float32(0))
        cox2 = colvec(cox2s, jnp.float32(0))
        coy2 = colvec(coy2s, jnp.float32(0))
        carea_s = jnp.maximum(cox2 - cox1, 0.0) * jnp.maximum(coy2 - coy1, 0.0)
        ix = jnp.maximum(jnp.minimum(kx2o, cox2) - jnp.maximum(kx1o, cox1), 0.0)
        iy = jnp.maximum(jnp.minimum(ky2o, coy2) - jnp.maximum(ky1o, coy1), 0.0)
        inter = ix * iy
        union = karea + carea_s - inter
        iou = inter / jnp.maximum(union, 1e-9)
        supp = jnp.max(iou, axis=1, keepdims=True) > _IOU_THR
        live_v = (counts < _KEEP) & (dones == 0) & (newdones == 0)
        keepit = jnp.logical_not(supp) & live_v
        cm = (lane_iota == counts) & keepit
        ks = jnp.where(cm, colvec(svals, jnp.float32(0)), ks)
        kl = jnp.where(cm, colvec(cs, jnp.int32(0)), kl)
        kbx1 = jnp.where(cm, colvec(px1s, jnp.float32(0)), kbx1)
        kby1 = jnp.where(cm, colvec(py1s, jnp.float32(0)), kby1)
        kbx2 = jnp.where(cm, colvec(px2s, jnp.float32(0)), kbx2)
        kby2 = jnp.where(cm, colvec(py2s, jnp.float32(0)), kby2)
        kx1o = jnp.where(cm, cox1, kx1o)
        ky1o = jnp.where(cm, coy1, ky1o)
        kx2o = jnp.where(cm, cox2, kx2o)
        ky2o = jnp.where(cm, coy2, ky2o)
        karea = jnp.where(cm, carea_s, karea)
        counts = counts + keepit.astype(jnp.int32)
        return (counts, newdones, ks, kl, kbx1, kby1, kbx2, kby2,
                kx1o, ky1o, kx2o, ky2o, karea)

    zf = jnp.zeros((_BS, 128), jnp.float32)
    zi = jnp.zeros((_BS, 128), jnp.int32)
    zc = jnp.zeros((_BS, 1), jnp.int32)
    carry0 = (zc, zc, zf, zi, zf, zf, zf, zf, zf, zf, zf, zf, zf)
    (counts, _, ks, kl, kbx1, kby1, kbx2, kby2,
     _, _, _, _, _) = jax.lax.while_loop(cond, body, carry0)

    first = lane_iota == 0
    padm = lane_iota >= counts

    def pad(vec, zero):
        slot0 = jnp.sum(jnp.where(first, vec, zero), axis=1, keepdims=True)
        return jnp.where(padm, slot0, vec)

    s_out_ref[...] = pad(ks, 0.0)[:, :_KEEP]
    l_out_ref[...] = pad(kl, 0)[:, :_KEEP]
    x1_out_ref[...] = pad(kbx1, 0.0)[:, :_KEEP]
    y1_out_ref[...] = pad(kby1, 0.0)[:, :_KEEP]
    x2_out_ref[...] = pad(kbx2, 0.0)[:, :_KEEP]
    y2_out_ref[...] = pad(kby2, 0.0)[:, :_KEEP]


def _build_call(interpret=False):
    f32 = jnp.float32
    return pl.pallas_call(
        _nms_body,
        out_shape=[
            jax.ShapeDtypeStruct((_BS, _KEEP), f32),
            jax.ShapeDtypeStruct((_BS, _KEEP), jnp.int32),
            jax.ShapeDtypeStruct((_BS, _KEEP), f32),
            jax.ShapeDtypeStruct((_BS, _KEEP), f32),
            jax.ShapeDtypeStruct((_BS, _KEEP), f32),
            jax.ShapeDtypeStruct((_BS, _KEEP), f32),
        ],
        scratch_shapes=[
            pltpu.VMEM((_BS * _NC, _NQ), jnp.int32),
            pltpu.VMEM((_BS * _NC, _NQ), f32),
            pltpu.VMEM((_NC, _BS), jnp.int32),
            pltpu.VMEM((_NC, _BS), jnp.int32),
        ],
        interpret=interpret,
    )


@jax.jit
def kernel(pred_logits, pred_boxes, target_sizes):
    logits_t = jnp.transpose(pred_logits, (0, 2, 1))
    boxes_t = jnp.transpose(pred_boxes, (0, 2, 1))
    img_h = target_sizes[:, 0].astype(jnp.float32)
    img_w = target_sizes[:, 1].astype(jnp.float32)
    scale = jnp.stack([img_w, img_h], axis=1)
    scores, labels, x1, y1, x2, y2 = _build_call()(
        logits_t, boxes_t, pred_boxes, scale)
    boxes = jnp.stack([x1, y1, x2, y2], axis=-1)
    return scores, labels, boxes


# speedup vs baseline: 28.0915x; 1.1177x over previous
import functools

import jax
import jax.numpy as jnp
from jax.experimental import pallas as pl
from jax.experimental.pallas import tpu as pltpu

_BS = 4
_NQ = 1000
_NC = 91
_PRE_TOPK = 10000
_KEEP = 100
_IOU_THR = 0.7
_BIG = 1 << 30
_KS, _KBX1, _KBY1, _KBX2, _KBY2, _KX1O, _KY1O, _KX2O, _KY2O, _KAREA = range(10)


def _nms_body(logits_ref, boxes_t_ref, boxes_raw_ref, scale_ref,
              s_out_ref, l_out_ref, x1_out_ref, y1_out_ref, x2_out_ref,
              y2_out_ref, msb_ref, s_ref, rmax_ref, rbest_ref, kf_ref, kl_ref):
    row_iota = jax.lax.broadcasted_iota(jnp.int32, (_NC, _NQ), 0)
    col_iota = jax.lax.broadcasted_iota(jnp.int32, (_NC, _NQ), 1)
    fidx = col_iota * _NC + row_iota
    riota = jax.lax.broadcasted_iota(jnp.int32, (_NC, 1), 0)
    col1 = jax.lax.broadcasted_iota(jnp.int32, (1, _NQ), 1)
    lane128 = jax.lax.broadcasted_iota(jnp.int32, (1, 128), 1)

    sbits_all = []
    sw_all = []
    sh_all = []
    for b in range(_BS):
        s_b = jax.nn.sigmoid(logits_ref[b])
        s_ref[b * _NC:(b + 1) * _NC, :] = s_b
        sbits_all.append(jax.lax.bitcast_convert_type(s_b, jnp.int32))
        sw_all.append(jnp.sum(scale_ref[b:b + 1, 0:1]))
        sh_all.append(jnp.sum(scale_ref[b:b + 1, 1:2]))

    kf_ref[...] = jnp.zeros((10 * _BS, 128), jnp.float32)
    kl_ref[...] = jnp.zeros((_BS, 128), jnp.int32)

    def bs_val(_, state):
        out = []
        for b in range(_BS):
            lo, hi = state[2 * b], state[2 * b + 1]
            mid = (lo + hi) // 2
            take_hi = jnp.sum((sbits_all[b] > mid).astype(jnp.int32)) >= _PRE_TOPK
            out.append(jnp.where(take_hi, mid, lo))
            out.append(jnp.where(take_hi, hi, mid))
        return tuple(out)

    st0 = (jnp.int32(-1), jnp.int32(0x3F800000)) * _BS
    st = jax.lax.fori_loop(0, 31, bs_val, st0)
    taus = [st[2 * b + 1] for b in range(_BS)]
    needs = [
        _PRE_TOPK - jnp.sum((sbits_all[b] > taus[b]).astype(jnp.int32))
        for b in range(_BS)
    ]
    ties = [sbits_all[b] == taus[b] for b in range(_BS)]

    def bs_idx(_, state):
        out = []
        for b in range(_BS):
            lo, hi = state[2 * b], state[2 * b + 1]
            mid = (lo + hi) // 2
            cnt = jnp.sum((ties[b] & (fidx < mid)).astype(jnp.int32))
            take_hi = cnt >= needs[b]
            out.append(jnp.where(take_hi, lo, mid))
            out.append(jnp.where(take_hi, mid, hi))
        return tuple(out)

    st0 = (jnp.int32(0), jnp.int32(_NQ * _NC)) * _BS
    st = jax.lax.fori_loop(0, 17, bs_idx, st0)
    mstars = [st[2 * b + 1] for b in range(_BS)]

    off_units = []
    for b in range(_BS):
        elig = (sbits_all[b] > taus[b]) | (ties[b] & (fidx < mstars[b]))
        msb0 = jnp.where(elig, sbits_all[b], jnp.int32(-1))
        msb_ref[b * _NC:(b + 1) * _NC, :] = msb0
        rmax0 = jnp.max(msb0, axis=1, keepdims=True)
        qmin0 = jnp.min(jnp.where(msb0 == rmax0, col_iota, _BIG),
                        axis=1, keepdims=True)
        rmax_ref[:, b:b + 1] = rmax0
        rbest_ref[:, b:b + 1] = qmin0 * _NC + riota
        cxt = boxes_t_ref[b, 0:1, :]
        cyt = boxes_t_ref[b, 1:2, :]
        wt = boxes_t_ref[b, 2:3, :]
        ht = boxes_t_ref[b, 3:4, :]
        x1t = (cxt - 0.5 * wt) * sw_all[b]
        y1t = (cyt - 0.5 * ht) * sh_all[b]
        x2t = (cxt + 0.5 * wt) * sw_all[b]
        y2t = (cyt + 0.5 * ht) * sh_all[b]
        qmax = jnp.maximum(jnp.maximum(x1t, x2t), jnp.maximum(y1t, y2t))
        row_any = jnp.any(elig, axis=0, keepdims=True)
        maxc = jnp.max(jnp.where(row_any, qmax, jnp.float32(-3.4e38)))
        off_units.append(maxc + 1.0)

    halfsign = jnp.concatenate(
        [jnp.full((1, 2), -0.5, jnp.float32), jnp.full((1, 2), 0.5, jnp.float32)],
        axis=1)

    def cond(carry):
        live = None
        for b in range(_BS):
            lb = (carry[2 * b] < _KEEP) & (carry[2 * b + 1] == 0)
            live = lb if live is None else (live | lb)
        return live

    def body(carry):
        rm = rmax_ref[...]
        mb = jnp.max(rm, axis=0, keepdims=True)
        fpick = jnp.min(jnp.where(rm == mb, rbest_ref[...], _BIG),
                        axis=0, keepdims=True)
        out = []
        for b in range(_BS):
            cnt_b, done_b = carry[2 * b], carry[2 * b + 1]
            f_b = jnp.sum(fpick[:, b:b + 1])
            mb_b = jnp.sum(mb[:, b:b + 1])
            live_b = (cnt_b < _KEEP) & (done_b == 0) & (mb_b >= 0)
            newdone_b = jnp.where(mb_b < 0, jnp.int32(1), done_b)
            q = f_b // _NC
            c = f_b % _NC
            r = b * _NC + c
            row = msb_ref[pl.ds(r, 1), :]
            row = jnp.where((col1 == q) & live_b, jnp.int32(-1), row)
            msb_ref[pl.ds(r, 1), :] = row
            nrmax = jnp.max(row, axis=1, keepdims=True)
            nqmin = jnp.min(jnp.where(row == nrmax, col1, _BIG),
                            axis=1, keepdims=True)
            rmax_ref[pl.ds(c, 1), b:b + 1] = nrmax
            rbest_ref[pl.ds(c, 1), b:b + 1] = nqmin * _NC + c
            srow = s_ref[pl.ds(r, 1), :]
            sval = jnp.sum(jnp.where(col1 == q, srow, 0.0),
                           axis=1, keepdims=True)
            braw = boxes_raw_ref[b, pl.ds(q, 1), :]
            cxy2 = jnp.concatenate([braw[:, 0:2], braw[:, 0:2]], axis=1)
            wh2 = jnp.concatenate([braw[:, 2:4], braw[:, 2:4]], axis=1)
            svec = scale_ref[b:b + 1, :]
            pxy = (cxy2 + halfsign * wh2) * svec
            co = pxy + c.astype(jnp.float32) * off_units[b]
            d = jnp.maximum(co[:, 2:4] - co[:, 0:2], 0.0)
            carea = d[:, 0:1] * d[:, 1:2]
            kx1o = kf_ref[10 * b + _KX1O:10 * b + _KX1O + 1, :]
            ky1o = kf_ref[10 * b + _KY1O:10 * b + _KY1O + 1, :]
            kx2o = kf_ref[10 * b + _KX2O:10 * b + _KX2O + 1, :]
            ky2o = kf_ref[10 * b + _KY2O:10 * b + _KY2O + 1, :]
            karea = kf_ref[10 * b + _KAREA:10 * b + _KAREA + 1, :]
            ix = jnp.maximum(
                jnp.minimum(kx2o, co[:, 2:3]) - jnp.maximum(kx1o, co[:, 0:1]),
                0.0)
            iy = jnp.maximum(
                jnp.minimum(ky2o, co[:, 3:4]) - jnp.maximum(ky1o, co[:, 1:2]),
                0.0)
            inter = ix * iy
            union = karea + carea - inter
            iou = inter / jnp.maximum(union, 1e-9)
            supp = jnp.max(iou) > _IOU_THR
            keep_b = jnp.logical_not(supp) & live_b
            cm = (lane128 == cnt_b) & keep_b
            upd_f = [(_KS, sval), (_KBX1, pxy[:, 0:1]), (_KBY1, pxy[:, 1:2]),
                     (_KBX2, pxy[:, 2:3]), (_KBY2, pxy[:, 3:4]),
                     (_KX1O, co[:, 0:1]), (_KY1O, co[:, 1:2]),
                     (_KX2O, co[:, 2:3]), (_KY2O, co[:, 3:4]), (_KAREA, carea)]
            for slot, val in upd_f:
                old = kf_ref[10 * b + slot:10 * b + slot + 1, :]
                kf_ref[10 * b + slot:10 * b + slot + 1, :] = (
                    jnp.where(cm, val, old))
            kl_old = kl_ref[b:b + 1, :]
            kl_ref[b:b + 1, :] = jnp.where(cm, c, kl_old)
            out.append(cnt_b + keep_b.astype(jnp.int32))
            out.append(newdone_b)
        return tuple(out)

    carry0 = (jnp.int32(0),) * (2 * _BS)
    final = jax.lax.while_loop(cond, body, carry0)

    for b in range(_BS):
        cnt_b = final[2 * b]
        padm = lane128 >= cnt_b

        def pad(vec):
            return jnp.where(padm, jnp.broadcast_to(vec[:, 0:1], vec.shape),
                             vec)

        s_out_ref[b:b + 1, :] = pad(kf_ref[10 * b + _KS:10 * b + _KS + 1, :])[:, :_KEEP]
        l_out_ref[b:b + 1, :] = pad(kl_ref[b:b + 1, :])[:, :_KEEP]
        x1_out_ref[b:b + 1, :] = pad(kf_ref[10 * b + _KBX1:10 * b + _KBX1 + 1, :])[:, :_KEEP]
        y1_out_ref[b:b + 1, :] = pad(kf_ref[10 * b + _KBY1:10 * b + _KBY1 + 1, :])[:, :_KEEP]
        x2_out_ref[b:b + 1, :] = pad(kf_ref[10 * b + _KBX2:10 * b + _KBX2 + 1, :])[:, :_KEEP]
        y2_out_ref[b:b + 1, :] = pad(kf_ref[10 * b + _KBY2:10 * b + _KBY2 + 1, :])[:, :_KEEP]


def _build_call(interpret=False):
    f32 = jnp.float32
    return pl.pallas_call(
        _nms_body,
        out_shape=[
            jax.ShapeDtypeStruct((_BS, _KEEP), f32),
            jax.ShapeDtypeStruct((_BS, _KEEP), jnp.int32),
            jax.ShapeDtypeStruct((_BS, _KEEP), f32),
            jax.ShapeDtypeStruct((_BS, _KEEP), f32),
            jax.ShapeDtypeStruct((_BS, _KEEP), f32),
            jax.ShapeDtypeStruct((_BS, _KEEP), f32),
        ],
        scratch_shapes=[
            pltpu.VMEM((_BS * _NC, _NQ), jnp.int32),
            pltpu.VMEM((_BS * _NC, _NQ), f32),
            pltpu.VMEM((_NC, _BS), jnp.int32),
            pltpu.VMEM((_NC, _BS), jnp.int32),
            pltpu.VMEM((10 * _BS, 128), f32),
            pltpu.VMEM((_BS, 128), jnp.int32),
        ],
        interpret=interpret,
    )


@jax.jit
def kernel(pred_logits, pred_boxes, target_sizes):
    logits_t = jnp.transpose(pred_logits, (0, 2, 1))
    boxes_t = jnp.transpose(pred_boxes, (0, 2, 1))
    img_h = target_sizes[:, 0].astype(jnp.float32)
    img_w = target_sizes[:, 1].astype(jnp.float32)
    scale = jnp.stack([img_w, img_h, img_w, img_h], axis=1)
    scores, labels, x1, y1, x2, y2 = _build_call()(
        logits_t, boxes_t, pred_boxes, scale)
    boxes = jnp.stack([x1, y1, x2, y2], axis=-1)
    return scores, labels, boxes
